# baseline (speedup 1.0000x reference)
import functools

import jax
import jax.numpy as jnp
from jax import lax
from jax.experimental import pallas as pl
from jax.experimental.pallas import tpu as pltpu
from jax.experimental.pallas import tpu_sc as plsc

NC = 2
NS = 16
L = 16


def _complex_score_kernel(B, D, CB):
    NW = NC * NS
    BPW = B // NW
    NCHUNK = BPW // CB
    NCH = D // L

    mesh = plsc.VectorSubcoreMesh(core_axis_name="c", subcore_axis_name="s")

    @functools.partial(
        pl.kernel,
        out_type=jax.ShapeDtypeStruct((B,), jnp.float32),
        mesh=mesh,
        compiler_params=pltpu.CompilerParams(
            needs_layout_passes=False, use_tc_tiling_on_sc=False),
        scratch_types=[
            pltpu.VMEM((BPW,), jnp.int32),
            pltpu.VMEM((BPW,), jnp.int32),
            pltpu.VMEM((BPW,), jnp.int32),
            pltpu.VMEM((CB, D), jnp.float32),
            pltpu.VMEM((CB, D), jnp.float32),
            pltpu.VMEM((CB, D), jnp.float32),
            pltpu.VMEM((CB, D), jnp.float32),
            pltpu.VMEM((CB, D), jnp.float32),
            pltpu.VMEM((CB, D), jnp.float32),
            pltpu.VMEM((BPW,), jnp.float32),
            pltpu.VMEM((L * L,), jnp.float32),
            pltpu.SemaphoreType.DMA,
        ],
    )
    def k(hidx_hbm, ridx_hbm, tidx_hbm, nre_hbm, nim_hbm, rre_hbm, rim_hbm,
          out_hbm, hidx_v, ridx_v, tidx_v, hr_v, hi_v, rr_v, ri_v, tr_v,
          ti_v, out_v, part_v, sem):
        wid = lax.axis_index("s") * NC + lax.axis_index("c")
        base = wid * BPW
        pltpu.sync_copy(hidx_hbm.at[pl.ds(base, BPW)], hidx_v)
        pltpu.sync_copy(ridx_hbm.at[pl.ds(base, BPW)], ridx_v)
        pltpu.sync_copy(tidx_hbm.at[pl.ds(base, BPW)], tidx_v)

        for c in range(NCHUNK):
            hslice = hidx_v.at[pl.ds(c * CB, CB)]
            rslice = ridx_v.at[pl.ds(c * CB, CB)]
            tslice = tidx_v.at[pl.ds(c * CB, CB)]
            cps = [
                pltpu.async_copy(nre_hbm.at[hslice], hr_v, sem),
                pltpu.async_copy(nim_hbm.at[hslice], hi_v, sem),
                pltpu.async_copy(rre_hbm.at[rslice], rr_v, sem),
                pltpu.async_copy(rim_hbm.at[rslice], ri_v, sem),
                pltpu.async_copy(nre_hbm.at[tslice], tr_v, sem),
                pltpu.async_copy(nim_hbm.at[tslice], ti_v, sem),
            ]
            for cp in cps:
                cp.wait()

            def group(g, carry):
                for j in range(L):
                    e = g * L + j
                    acc = jnp.zeros((L,), jnp.float32)
                    for ch in range(NCH):
                        sl = pl.ds(ch * L, L)
                        hr = hr_v[e, sl]
                        hi = hi_v[e, sl]
                        rr = rr_v[e, sl]
                        ri = ri_v[e, sl]
                        tr = tr_v[e, sl]
                        ti = ti_v[e, sl]
                        acc = acc + rr * (hr * tr + hi * ti) \
                                  + ri * (hr * ti - hi * tr)
                    part_v[pl.ds(j * L, L)] = acc
                lane = lax.iota(jnp.int32, L)
                scores = jnp.zeros((L,), jnp.float32)
                for j in range(L):
                    col = plsc.load_gather(part_v, [lane * L + j])
                    scores = scores + col
                gofs = pl.multiple_of(g * L, L)
                out_v[pl.ds(c * CB + gofs, L)] = scores
                return carry

            lax.fori_loop(0, CB // L, group, 0)

        pltpu.sync_copy(out_v, out_hbm.at[pl.ds(base, BPW)])

    return k


def kernel(head_index, rel_type, tail_index, node_emb, node_emb_im,
           rel_emb, rel_emb_im):
    B = head_index.shape[0]
    D = node_emb.shape[1]
    k = _complex_score_kernel(B, D, CB=128)
    return k(head_index.astype(jnp.int32), rel_type.astype(jnp.int32),
             tail_index.astype(jnp.int32), node_emb, node_emb_im,
             rel_emb, rel_emb_im)
